# Initial kernel scaffold; baseline (speedup 1.0000x reference)
import jax
import jax.numpy as jnp
from jax.experimental import pallas as pl


def kernel(hidden_states, gate_w, w_gate, w_up, w_down, num_global_tokens, max_num_tokens_per_gpu):
    raise NotImplementedError("write your pallas kernel here")


# speedup vs baseline: 2.3629x; 2.3629x over previous
import functools

import jax
import jax.numpy as jnp
from jax.experimental import pallas as pl
from jax.experimental.pallas import tpu as pltpu

E = 16
TOP_K = 2
HIDDEN = 1024
INTER = 512
T = 2048
NEG_INF = float("-inf")


def _moe_dense_body(x_ref, gate_ref, wg_ref, wu_ref, wd_ref, out_ref,
                    comb_ref, acc_ref):
    e = pl.program_id(0)
    x = x_ref[...]

    @pl.when(e == 0)
    def _router():
        logits = jax.lax.dot_general(
            x, gate_ref[...], (((1,), (1,)), ((), ())),
            preferred_element_type=jnp.float32)
        ii = jax.lax.broadcasted_iota(jnp.int32, (T, E), 1)
        m1 = jnp.max(logits, axis=-1, keepdims=True)
        i1 = jnp.min(jnp.where(logits == m1, ii, E), axis=-1, keepdims=True)
        l2 = jnp.where(ii == i1, NEG_INF, logits)
        m2 = jnp.max(l2, axis=-1, keepdims=True)
        i2 = jnp.min(jnp.where(l2 == m2, ii, E), axis=-1, keepdims=True)
        r = jnp.exp(m2 - m1)
        w1 = 1.0 / (1.0 + r)
        w2 = 1.0 - w1
        comb_ref[...] = jnp.where(ii == i1, w1, 0.0) + jnp.where(ii == i2, w2, 0.0)

    wg = wg_ref[0]
    wu = wu_ref[0]
    wd = wd_ref[0]
    g = jax.lax.dot_general(x, wg, (((1,), (1,)), ((), ())),
                            preferred_element_type=jnp.float32)
    u = jax.lax.dot_general(x, wu, (((1,), (1,)), ((), ())),
                            preferred_element_type=jnp.float32)
    h = (g * jax.nn.sigmoid(g)) * u
    y = jax.lax.dot_general(h, wd, (((1,), (1,)), ((), ())),
                            preferred_element_type=jnp.float32)

    ii = jax.lax.broadcasted_iota(jnp.int32, (T, E), 1)
    w = jnp.sum(jnp.where(ii == e, comb_ref[...], 0.0), axis=1, keepdims=True)

    @pl.when(e == 0)
    def _init():
        acc_ref[...] = w * y

    @pl.when(e > 0)
    def _acc():
        acc_ref[...] = acc_ref[...] + w * y

    @pl.when(e == E - 1)
    def _fin():
        out_ref[...] = acc_ref[...]


def kernel(hidden_states, gate_w, w_gate, w_up, w_down, num_global_tokens,
           max_num_tokens_per_gpu):
    del num_global_tokens, max_num_tokens_per_gpu
    out = pl.pallas_call(
        _moe_dense_body,
        grid=(E,),
        in_specs=[
            pl.BlockSpec((T, HIDDEN), lambda e: (0, 0)),
            pl.BlockSpec((E, HIDDEN), lambda e: (0, 0)),
            pl.BlockSpec((1, INTER, HIDDEN), lambda e: (e, 0, 0)),
            pl.BlockSpec((1, INTER, HIDDEN), lambda e: (e, 0, 0)),
            pl.BlockSpec((1, HIDDEN, INTER), lambda e: (e, 0, 0)),
        ],
        out_specs=pl.BlockSpec((T, HIDDEN), lambda e: (0, 0)),
        out_shape=jax.ShapeDtypeStruct((T, HIDDEN), jnp.float32),
        scratch_shapes=[
            pltpu.VMEM((T, E), jnp.float32),
            pltpu.VMEM((T, HIDDEN), jnp.float32),
        ],
    )(hidden_states, gate_w, w_gate, w_up, w_down)
    return out


# speedup vs baseline: 2.7336x; 1.1569x over previous
import functools

import jax
import jax.numpy as jnp
from jax import lax
from jax.experimental import pallas as pl
from jax.experimental.pallas import tpu as pltpu
from jax.experimental.pallas import tpu_sc as plsc

E = 16
TOP_K = 2
HIDDEN = 1024
INTER = 512
T = 2048
NEG_INF = float("-inf")

BLK = 256
NB = (T * TOP_K) // BLK + (E - 1)
GROUP_ROWS = NB * BLK
CHUNK = 256


def _router_body(x_ref, gate_ref, dest_ref, meta_ref, wts_ref):
    x = x_ref[...]
    logits = lax.dot_general(x, gate_ref[...], (((1,), (1,)), ((), ())),
                             preferred_element_type=jnp.float32)
    ii = lax.broadcasted_iota(jnp.int32, (T, E), 1)
    m1 = jnp.max(logits, axis=-1, keepdims=True)
    i1 = jnp.min(jnp.where(logits == m1, ii, E), axis=-1, keepdims=True)
    l2 = jnp.where(ii == i1, NEG_INF, logits)
    m2 = jnp.max(l2, axis=-1, keepdims=True)
    i2 = jnp.min(jnp.where(l2 == m2, ii, E), axis=-1, keepdims=True)
    r = jnp.exp(m2 - m1)
    w1 = 1.0 / (1.0 + r)
    w2 = 1.0 - w1
    wts_ref[0, :] = w1[:, 0]
    wts_ref[1, :] = w2[:, 0]

    oh0 = (ii == i1).astype(jnp.float32)
    oh1 = (ii == i2).astype(jnp.float32)

    rr = lax.broadcasted_iota(jnp.int32, (CHUNK, CHUNK), 0)
    cc = lax.broadcasted_iota(jnp.int32, (CHUNK, CHUNK), 1)
    ltri = (rr > cc).astype(jnp.float32)

    def _excl_cumsum(oh):
        parts = []
        off = jnp.zeros((1, E), jnp.float32)
        for c in range(T // CHUNK):
            blk = oh[c * CHUNK:(c + 1) * CHUNK, :]
            exc = lax.dot_general(ltri, blk, (((1,), (0,)), ((), ())),
                                  preferred_element_type=jnp.float32)
            parts.append(exc + off)
            off = off + jnp.sum(blk, axis=0, keepdims=True)
        return jnp.concatenate(parts, axis=0), off

    exc0, cnt0 = _excl_cumsum(oh0)
    exc1, cnt1 = _excl_cumsum(oh1)
    counts = cnt0 + cnt1
    nb = jnp.floor((counts + (BLK - 1)) / BLK)
    er = lax.broadcasted_iota(jnp.int32, (E, E), 0)
    ec = lax.broadcasted_iota(jnp.int32, (E, E), 1)
    eutri = (er < ec).astype(jnp.float32)
    bs = lax.dot_general(nb, eutri, (((1,), (0,)), ((), ())),
                         preferred_element_type=jnp.float32)
    total = jnp.sum(nb)

    rank0 = jnp.sum(oh0 * exc0, axis=1)
    rank1 = jnp.sum(oh1 * (cnt0 + exc1), axis=1)
    base0 = jnp.sum(oh0 * bs, axis=1) * BLK
    base1 = jnp.sum(oh1 * bs, axis=1) * BLK
    dest_ref[0, :] = (base0 + rank0).astype(jnp.int32)
    dest_ref[1, :] = (base1 + rank1).astype(jnp.int32)

    bb = lax.broadcasted_iota(jnp.int32, (64, E), 0).astype(jnp.float32)
    emap = jnp.sum((bs <= bb).astype(jnp.int32), axis=1) - 1
    bidx = lax.broadcasted_iota(jnp.int32, (64, 1), 0).astype(jnp.float32)[:, 0]
    active = (bidx < total).astype(jnp.int32)
    meta_ref[0, :] = jnp.clip(emap, 0, E - 1)
    meta_ref[1, :] = active


def _router_call(x, gate_w):
    return pl.pallas_call(
        _router_body,
        out_shape=(
            jax.ShapeDtypeStruct((2, T), jnp.int32),
            jax.ShapeDtypeStruct((2, 64), jnp.int32),
            jax.ShapeDtypeStruct((2, T), jnp.float32),
        ),
    )(x, gate_w)


def _make_sc_scatter():
    mesh = plsc.VectorSubcoreMesh(core_axis_name="c", subcore_axis_name="s")
    n_workers = 32
    tpw = T // n_workers

    @functools.partial(
        pl.kernel, mesh=mesh,
        out_type=jax.ShapeDtypeStruct((GROUP_ROWS, HIDDEN), jnp.float32),
        scratch_types=[
            pltpu.VMEM((tpw, HIDDEN), jnp.float32),
            pltpu.VMEM((tpw,), jnp.int32),
            pltpu.VMEM((tpw,), jnp.int32),
            pltpu.SemaphoreType.DMA,
        ],
    )
    def k(x_hbm, dest_hbm, xg_hbm, rows_v, idx0_v, idx1_v, sem):
        wid = lax.axis_index("s") * 2 + lax.axis_index("c")
        base = wid * tpw
        pltpu.sync_copy(x_hbm.at[pl.ds(base, tpw)], rows_v)
        pltpu.sync_copy(dest_hbm.at[0, pl.ds(base, tpw)], idx0_v)
        pltpu.sync_copy(dest_hbm.at[1, pl.ds(base, tpw)], idx1_v)
        pltpu.async_copy(rows_v, xg_hbm.at[idx0_v], sem).wait()
        pltpu.async_copy(rows_v, xg_hbm.at[idx1_v], sem).wait()

    return k


def _ffn_body(s_ref, xg_ref, wg_ref, wu_ref, wd_ref, yg_ref):
    b = pl.program_id(0)

    @pl.when(s_ref[1, b] == 1)
    def _compute():
        xb = xg_ref[...]
        g = lax.dot_general(xb, wg_ref[0], (((1,), (1,)), ((), ())),
                            preferred_element_type=jnp.float32)
        u = lax.dot_general(xb, wu_ref[0], (((1,), (1,)), ((), ())),
                            preferred_element_type=jnp.float32)
        h = (g * jax.nn.sigmoid(g)) * u
        yg_ref[...] = lax.dot_general(h, wd_ref[0], (((1,), (1,)), ((), ())),
                                      preferred_element_type=jnp.float32)


def _ffn_call(meta, xg, w_gate, w_up, w_down):
    grid_spec = pltpu.PrefetchScalarGridSpec(
        num_scalar_prefetch=1,
        grid=(NB,),
        in_specs=[
            pl.BlockSpec((BLK, HIDDEN), lambda b, s: (b, 0)),
            pl.BlockSpec((1, INTER, HIDDEN), lambda b, s: (s[0, b], 0, 0)),
            pl.BlockSpec((1, INTER, HIDDEN), lambda b, s: (s[0, b], 0, 0)),
            pl.BlockSpec((1, HIDDEN, INTER), lambda b, s: (s[0, b], 0, 0)),
        ],
        out_specs=pl.BlockSpec((BLK, HIDDEN), lambda b, s: (b, 0)),
    )
    return pl.pallas_call(
        _ffn_body,
        grid_spec=grid_spec,
        out_shape=jax.ShapeDtypeStruct((GROUP_ROWS, HIDDEN), jnp.float32),
    )(meta, xg, w_gate, w_up, w_down)


def _make_sc_gather():
    mesh = plsc.VectorSubcoreMesh(core_axis_name="c", subcore_axis_name="s")
    n_workers = 32
    tpw = T // n_workers

    @functools.partial(
        pl.kernel, mesh=mesh,
        out_type=(
            jax.ShapeDtypeStruct((T, HIDDEN), jnp.float32),
            jax.ShapeDtypeStruct((T, HIDDEN), jnp.float32),
        ),
        scratch_types=[
            pltpu.VMEM((tpw, HIDDEN), jnp.float32),
            pltpu.VMEM((tpw,), jnp.int32),
            pltpu.SemaphoreType.DMA,
        ],
    )
    def k(yg_hbm, dest_hbm, y0_hbm, y1_hbm, buf_v, idx_v, sem):
        wid = lax.axis_index("s") * 2 + lax.axis_index("c")
        base = wid * tpw
        pltpu.sync_copy(dest_hbm.at[0, pl.ds(base, tpw)], idx_v)
        pltpu.async_copy(yg_hbm.at[idx_v], buf_v, sem).wait()
        pltpu.sync_copy(buf_v, y0_hbm.at[pl.ds(base, tpw)])
        pltpu.sync_copy(dest_hbm.at[1, pl.ds(base, tpw)], idx_v)
        pltpu.async_copy(yg_hbm.at[idx_v], buf_v, sem).wait()
        pltpu.sync_copy(buf_v, y1_hbm.at[pl.ds(base, tpw)])

    return k


def _combine_body(w_ref, y0_ref, y1_ref, out_ref):
    w0 = w_ref[0, :][:, None]
    w1 = w_ref[1, :][:, None]
    out_ref[...] = w0 * y0_ref[...] + w1 * y1_ref[...]


def _combine_call(wts, y0, y1):
    return pl.pallas_call(
        _combine_body,
        out_shape=jax.ShapeDtypeStruct((T, HIDDEN), jnp.float32),
    )(wts, y0, y1)


def kernel(hidden_states, gate_w, w_gate, w_up, w_down, num_global_tokens,
           max_num_tokens_per_gpu):
    del num_global_tokens, max_num_tokens_per_gpu
    dest, meta, wts = _router_call(hidden_states, gate_w)
    xg = _make_sc_scatter()(hidden_states, dest)
    yg = _ffn_call(meta, xg, w_gate, w_up, w_down)
    y0, y1 = _make_sc_gather()(yg, dest)
    return _combine_call(wts, y0, y1)


# speedup vs baseline: 2.9144x; 1.0661x over previous
import functools

import jax
import jax.numpy as jnp
from jax import lax
from jax.experimental import pallas as pl
from jax.experimental.pallas import tpu as pltpu
from jax.experimental.pallas import tpu_sc as plsc

E = 16
TOP_K = 2
HIDDEN = 1024
INTER = 512
T = 2048
NEG_INF = float("-inf")

BLK = 256
NB = (T * TOP_K) // BLK + (E - 1)
GROUP_ROWS = NB * BLK
CHUNK = 256
NW = 32
TPW = T // NW


def _router_body(x_ref, gate_ref, dest_ref, meta_ref, wrep_ref):
    x = x_ref[...]
    logits = lax.dot_general(x, gate_ref[...], (((1,), (1,)), ((), ())),
                             preferred_element_type=jnp.float32)
    ii = lax.broadcasted_iota(jnp.int32, (T, E), 1)
    m1 = jnp.max(logits, axis=-1, keepdims=True)
    i1 = jnp.min(jnp.where(logits == m1, ii, E), axis=-1, keepdims=True)
    l2 = jnp.where(ii == i1, NEG_INF, logits)
    m2 = jnp.max(l2, axis=-1, keepdims=True)
    i2 = jnp.min(jnp.where(l2 == m2, ii, E), axis=-1, keepdims=True)
    r = jnp.exp(m2 - m1)
    w1 = 1.0 / (1.0 + r)
    w2 = 1.0 - w1
    wrep_ref[0:T, :] = jnp.broadcast_to(w1, (T, 128))
    wrep_ref[T:2 * T, :] = jnp.broadcast_to(w2, (T, 128))

    oh0 = (ii == i1).astype(jnp.float32)
    oh1 = (ii == i2).astype(jnp.float32)

    rr = lax.broadcasted_iota(jnp.int32, (CHUNK, CHUNK), 0)
    cc = lax.broadcasted_iota(jnp.int32, (CHUNK, CHUNK), 1)
    ltri = (rr > cc).astype(jnp.float32)

    def _excl_cumsum(oh):
        parts = []
        off = jnp.zeros((1, E), jnp.float32)
        for c in range(T // CHUNK):
            blk = oh[c * CHUNK:(c + 1) * CHUNK, :]
            exc = lax.dot_general(ltri, blk, (((1,), (0,)), ((), ())),
                                  preferred_element_type=jnp.float32)
            parts.append(exc + off)
            off = off + jnp.sum(blk, axis=0, keepdims=True)
        return jnp.concatenate(parts, axis=0), off

    exc0, cnt0 = _excl_cumsum(oh0)
    exc1, cnt1 = _excl_cumsum(oh1)
    counts = cnt0 + cnt1
    nb = jnp.floor((counts + (BLK - 1)) / BLK)
    er = lax.broadcasted_iota(jnp.int32, (E, E), 0)
    ec = lax.broadcasted_iota(jnp.int32, (E, E), 1)
    eutri = (er < ec).astype(jnp.float32)
    bs = lax.dot_general(nb, eutri, (((1,), (0,)), ((), ())),
                         preferred_element_type=jnp.float32)
    total = jnp.sum(nb)

    rank0 = jnp.sum(oh0 * exc0, axis=1)
    rank1 = jnp.sum(oh1 * (cnt0 + exc1), axis=1)
    base0 = jnp.sum(oh0 * bs, axis=1) * BLK
    base1 = jnp.sum(oh1 * bs, axis=1) * BLK
    dest_ref[0, :] = (base0 + rank0).astype(jnp.int32)
    dest_ref[1, :] = (base1 + rank1).astype(jnp.int32)

    bb = lax.broadcasted_iota(jnp.int32, (64, E), 0).astype(jnp.float32)
    emap = jnp.sum((bs <= bb).astype(jnp.int32), axis=1) - 1
    bidx = lax.broadcasted_iota(jnp.int32, (64, 1), 0).astype(jnp.float32)[:, 0]
    active = (bidx < total).astype(jnp.int32)
    meta_ref[0, :] = jnp.clip(emap, 0, E - 1)
    meta_ref[1, :] = active
    meta_ref[2, :] = jnp.minimum(bidx, total - 1.0).astype(jnp.int32)


def _router_call(x, gate_w):
    return pl.pallas_call(
        _router_body,
        out_shape=(
            jax.ShapeDtypeStruct((2, T), jnp.int32),
            jax.ShapeDtypeStruct((3, 64), jnp.int32),
            jax.ShapeDtypeStruct((2 * T, 128), jnp.float32),
        ),
    )(x, gate_w)


def _make_sc_scatter():
    mesh = plsc.VectorSubcoreMesh(core_axis_name="c", subcore_axis_name="s")

    @functools.partial(
        pl.kernel, mesh=mesh,
        out_type=(
            jax.ShapeDtypeStruct((GROUP_ROWS, HIDDEN), jnp.float32),
            jax.ShapeDtypeStruct((GROUP_ROWS, 128), jnp.float32),
        ),
        scratch_types=[
            pltpu.VMEM((TPW, HIDDEN), jnp.float32),
            pltpu.VMEM((TPW, 128), jnp.float32),
            pltpu.VMEM((TPW, 128), jnp.float32),
            pltpu.VMEM((TPW,), jnp.int32),
            pltpu.VMEM((TPW,), jnp.int32),
            pltpu.SemaphoreType.DMA,
            pltpu.SemaphoreType.DMA,
        ],
    )
    def k(x_hbm, dest_hbm, wrep_hbm, xg_hbm, wgrp_hbm,
          rows_v, w0_v, w1_v, idx0_v, idx1_v, sem_a, sem_b):
        wid = lax.axis_index("s") * 2 + lax.axis_index("c")
        base = wid * TPW
        pltpu.sync_copy(dest_hbm.at[0, pl.ds(base, TPW)], idx0_v)
        pltpu.sync_copy(dest_hbm.at[1, pl.ds(base, TPW)], idx1_v)
        pltpu.sync_copy(x_hbm.at[pl.ds(base, TPW)], rows_v)
        pltpu.sync_copy(wrep_hbm.at[pl.ds(base, TPW)], w0_v)
        pltpu.sync_copy(wrep_hbm.at[pl.ds(T + base, TPW)], w1_v)
        c0 = pltpu.async_copy(rows_v, xg_hbm.at[idx0_v], sem_a)
        c1 = pltpu.async_copy(rows_v, xg_hbm.at[idx1_v], sem_b)
        c0.wait()
        c1.wait()
        c2 = pltpu.async_copy(w0_v, wgrp_hbm.at[idx0_v], sem_a)
        c3 = pltpu.async_copy(w1_v, wgrp_hbm.at[idx1_v], sem_b)
        c2.wait()
        c3.wait()

    return k


def _ffn_body(s_ref, xg_ref, wgrp_ref, wg_ref, wu_ref, wd_ref, yg_ref):
    b = pl.program_id(0)

    @pl.when(s_ref[1, b] == 1)
    def _compute():
        xb = xg_ref[...]
        g = lax.dot_general(xb, wg_ref[0], (((1,), (1,)), ((), ())),
                            preferred_element_type=jnp.float32)
        u = lax.dot_general(xb, wu_ref[0], (((1,), (1,)), ((), ())),
                            preferred_element_type=jnp.float32)
        h = (g * jax.nn.sigmoid(g)) * u
        y = lax.dot_general(h, wd_ref[0], (((1,), (1,)), ((), ())),
                            preferred_element_type=jnp.float32)
        yg_ref[...] = y * wgrp_ref[:, 0:1]


def _ffn_call(meta, xg, wgrp, w_gate, w_up, w_down):
    grid_spec = pltpu.PrefetchScalarGridSpec(
        num_scalar_prefetch=1,
        grid=(NB,),
        in_specs=[
            pl.BlockSpec((BLK, HIDDEN), lambda b, s: (s[2, b], 0)),
            pl.BlockSpec((BLK, 128), lambda b, s: (s[2, b], 0)),
            pl.BlockSpec((1, INTER, HIDDEN), lambda b, s: (s[0, b], 0, 0)),
            pl.BlockSpec((1, INTER, HIDDEN), lambda b, s: (s[0, b], 0, 0)),
            pl.BlockSpec((1, HIDDEN, INTER), lambda b, s: (s[0, b], 0, 0)),
        ],
        out_specs=pl.BlockSpec((BLK, HIDDEN), lambda b, s: (s[2, b], 0)),
    )
    return pl.pallas_call(
        _ffn_body,
        grid_spec=grid_spec,
        out_shape=jax.ShapeDtypeStruct((GROUP_ROWS, HIDDEN), jnp.float32),
    )(meta, xg, wgrp, w_gate, w_up, w_down)


def _make_sc_combine():
    mesh = plsc.VectorSubcoreMesh(core_axis_name="c", subcore_axis_name="s")

    @functools.partial(
        pl.kernel, mesh=mesh,
        out_type=jax.ShapeDtypeStruct((T, HIDDEN), jnp.float32),
        scratch_types=[
            pltpu.VMEM((TPW // 2, HIDDEN), jnp.float32),
            pltpu.VMEM((TPW // 2, HIDDEN), jnp.float32),
            pltpu.VMEM((TPW // 2,), jnp.int32),
            pltpu.VMEM((TPW // 2,), jnp.int32),
            pltpu.SemaphoreType.DMA,
            pltpu.SemaphoreType.DMA,
        ],
    )
    def k(yg_hbm, dest_hbm, out_hbm, buf0_v, buf1_v, idx0_v, idx1_v,
          sem_a, sem_b):
        wid = lax.axis_index("s") * 2 + lax.axis_index("c")
        half = TPW // 2
        for h in range(2):
            base = wid * TPW + h * half
            pltpu.sync_copy(dest_hbm.at[0, pl.ds(base, half)], idx0_v)
            pltpu.sync_copy(dest_hbm.at[1, pl.ds(base, half)], idx1_v)
            c0 = pltpu.async_copy(yg_hbm.at[idx0_v], buf0_v, sem_a)
            c1 = pltpu.async_copy(yg_hbm.at[idx1_v], buf1_v, sem_b)
            c0.wait()
            c1.wait()

            def row(i, carry):
                for j in range(HIDDEN // 16):
                    sl = pl.ds(j * 16, 16)
                    buf0_v[i, sl] = buf0_v[i, sl] + buf1_v[i, sl]
                return carry

            lax.fori_loop(0, half, row, 0)
            pltpu.sync_copy(buf0_v, out_hbm.at[pl.ds(base, half)])

    return k


def kernel(hidden_states, gate_w, w_gate, w_up, w_down, num_global_tokens,
           max_num_tokens_per_gpu):
    del num_global_tokens, max_num_tokens_per_gpu
    dest, meta, wrep = _router_call(hidden_states, gate_w)
    xg, wgrp = _make_sc_scatter()(hidden_states, dest, wrep)
    yg = _ffn_call(meta, xg, wgrp, w_gate, w_up, w_down)
    return _make_sc_combine()(yg, dest)


# speedup vs baseline: 3.0635x; 1.0512x over previous
import functools

import jax
import jax.numpy as jnp
from jax import lax
from jax.experimental import pallas as pl
from jax.experimental.pallas import tpu as pltpu
from jax.experimental.pallas import tpu_sc as plsc

E = 16
TOP_K = 2
HIDDEN = 1024
INTER = 512
T = 2048
NEG_INF = float("-inf")

BLK = 512
NB = (T * TOP_K) // BLK + (E - 1)
GROUP_ROWS = NB * BLK
CHUNK = 256
NW = 32
TPW = T // NW


def _router_body(x_ref, gate_ref, dest_ref, meta_ref, wrep_ref):
    x = x_ref[...]
    logits = lax.dot_general(x, gate_ref[...], (((1,), (1,)), ((), ())),
                             preferred_element_type=jnp.float32)
    ii = lax.broadcasted_iota(jnp.int32, (T, E), 1)
    m1 = jnp.max(logits, axis=-1, keepdims=True)
    i1 = jnp.min(jnp.where(logits == m1, ii, E), axis=-1, keepdims=True)
    l2 = jnp.where(ii == i1, NEG_INF, logits)
    m2 = jnp.max(l2, axis=-1, keepdims=True)
    i2 = jnp.min(jnp.where(l2 == m2, ii, E), axis=-1, keepdims=True)
    r = jnp.exp(m2 - m1)
    w1 = 1.0 / (1.0 + r)
    w2 = 1.0 - w1
    wrep_ref[0:T, :] = jnp.broadcast_to(w1, (T, 128))
    wrep_ref[T:2 * T, :] = jnp.broadcast_to(w2, (T, 128))

    oh0 = (ii == i1).astype(jnp.float32)
    oh1 = (ii == i2).astype(jnp.float32)

    rr = lax.broadcasted_iota(jnp.int32, (CHUNK, CHUNK), 0)
    cc = lax.broadcasted_iota(jnp.int32, (CHUNK, CHUNK), 1)
    ltri = (rr > cc).astype(jnp.float32)

    def _excl_cumsum(oh):
        parts = []
        off = jnp.zeros((1, E), jnp.float32)
        for c in range(T // CHUNK):
            blk = oh[c * CHUNK:(c + 1) * CHUNK, :]
            exc = lax.dot_general(ltri, blk, (((1,), (0,)), ((), ())),
                                  preferred_element_type=jnp.float32)
            parts.append(exc + off)
            off = off + jnp.sum(blk, axis=0, keepdims=True)
        return jnp.concatenate(parts, axis=0), off

    exc0, cnt0 = _excl_cumsum(oh0)
    exc1, cnt1 = _excl_cumsum(oh1)
    counts = cnt0 + cnt1
    nb = jnp.floor((counts + (BLK - 1)) / BLK)
    er = lax.broadcasted_iota(jnp.int32, (E, E), 0)
    ec = lax.broadcasted_iota(jnp.int32, (E, E), 1)
    eutri = (er < ec).astype(jnp.float32)
    bs = lax.dot_general(nb, eutri, (((1,), (0,)), ((), ())),
                         preferred_element_type=jnp.float32)
    total = jnp.sum(nb)

    rank0 = jnp.sum(oh0 * exc0, axis=1)
    rank1 = jnp.sum(oh1 * (cnt0 + exc1), axis=1)
    base0 = jnp.sum(oh0 * bs, axis=1) * BLK
    base1 = jnp.sum(oh1 * bs, axis=1) * BLK
    dest_ref[0, :] = (base0 + rank0).astype(jnp.int32)
    dest_ref[1, :] = (base1 + rank1).astype(jnp.int32)

    bb = lax.broadcasted_iota(jnp.int32, (64, E), 0).astype(jnp.float32)
    emap = jnp.sum((bs <= bb).astype(jnp.int32), axis=1) - 1
    bidx = lax.broadcasted_iota(jnp.int32, (64, 1), 0).astype(jnp.float32)[:, 0]
    active = (bidx < total).astype(jnp.int32)
    meta_ref[0, :] = jnp.clip(emap, 0, E - 1)
    meta_ref[1, :] = active
    meta_ref[2, :] = jnp.minimum(bidx, total - 1.0).astype(jnp.int32)


def _router_call(x, gate_w):
    return pl.pallas_call(
        _router_body,
        out_shape=(
            jax.ShapeDtypeStruct((2, T), jnp.int32),
            jax.ShapeDtypeStruct((3, 64), jnp.int32),
            jax.ShapeDtypeStruct((2 * T, 128), jnp.float32),
        ),
    )(x, gate_w)


def _make_sc_scatter():
    mesh = plsc.VectorSubcoreMesh(core_axis_name="c", subcore_axis_name="s")

    @functools.partial(
        pl.kernel, mesh=mesh,
        out_type=(
            jax.ShapeDtypeStruct((GROUP_ROWS, HIDDEN), jnp.float32),
            jax.ShapeDtypeStruct((GROUP_ROWS, 128), jnp.float32),
        ),
        scratch_types=[
            pltpu.VMEM((TPW, HIDDEN), jnp.float32),
            pltpu.VMEM((TPW, 128), jnp.float32),
            pltpu.VMEM((TPW, 128), jnp.float32),
            pltpu.VMEM((TPW,), jnp.int32),
            pltpu.VMEM((TPW,), jnp.int32),
            pltpu.SemaphoreType.DMA,
            pltpu.SemaphoreType.DMA,
        ],
    )
    def k(x_hbm, dest_hbm, wrep_hbm, xg_hbm, wgrp_hbm,
          rows_v, w0_v, w1_v, idx0_v, idx1_v, sem_a, sem_b):
        wid = lax.axis_index("s") * 2 + lax.axis_index("c")
        base = wid * TPW
        pltpu.sync_copy(dest_hbm.at[0, pl.ds(base, TPW)], idx0_v)
        pltpu.sync_copy(dest_hbm.at[1, pl.ds(base, TPW)], idx1_v)
        pltpu.sync_copy(x_hbm.at[pl.ds(base, TPW)], rows_v)
        pltpu.sync_copy(wrep_hbm.at[pl.ds(base, TPW)], w0_v)
        pltpu.sync_copy(wrep_hbm.at[pl.ds(T + base, TPW)], w1_v)
        c0 = pltpu.async_copy(rows_v, xg_hbm.at[idx0_v], sem_a)
        c1 = pltpu.async_copy(rows_v, xg_hbm.at[idx1_v], sem_b)
        c0.wait()
        c1.wait()
        c2 = pltpu.async_copy(w0_v, wgrp_hbm.at[idx0_v], sem_a)
        c3 = pltpu.async_copy(w1_v, wgrp_hbm.at[idx1_v], sem_b)
        c2.wait()
        c3.wait()

    return k


def _ffn_body(s_ref, xg_ref, wgrp_ref, wg_ref, wu_ref, wd_ref, yg_ref):
    b = pl.program_id(0)

    @pl.when(s_ref[1, b] == 1)
    def _compute():
        xb = xg_ref[...]
        g = lax.dot_general(xb, wg_ref[0], (((1,), (1,)), ((), ())),
                            preferred_element_type=jnp.float32)
        u = lax.dot_general(xb, wu_ref[0], (((1,), (1,)), ((), ())),
                            preferred_element_type=jnp.float32)
        h = (g * jax.nn.sigmoid(g)) * u
        y = lax.dot_general(h, wd_ref[0], (((1,), (1,)), ((), ())),
                            preferred_element_type=jnp.float32)
        yg_ref[...] = y * wgrp_ref[:, 0:1]


def _ffn_call(meta, xg, wgrp, w_gate, w_up, w_down):
    grid_spec = pltpu.PrefetchScalarGridSpec(
        num_scalar_prefetch=1,
        grid=(NB,),
        in_specs=[
            pl.BlockSpec((BLK, HIDDEN), lambda b, s: (s[2, b], 0)),
            pl.BlockSpec((BLK, 128), lambda b, s: (s[2, b], 0)),
            pl.BlockSpec((1, INTER, HIDDEN), lambda b, s: (s[0, b], 0, 0)),
            pl.BlockSpec((1, INTER, HIDDEN), lambda b, s: (s[0, b], 0, 0)),
            pl.BlockSpec((1, HIDDEN, INTER), lambda b, s: (s[0, b], 0, 0)),
        ],
        out_specs=pl.BlockSpec((BLK, HIDDEN), lambda b, s: (s[2, b], 0)),
    )
    return pl.pallas_call(
        _ffn_body,
        grid_spec=grid_spec,
        out_shape=jax.ShapeDtypeStruct((GROUP_ROWS, HIDDEN), jnp.float32),
    )(meta, xg, wgrp, w_gate, w_up, w_down)


def _make_sc_combine():
    mesh = plsc.VectorSubcoreMesh(core_axis_name="c", subcore_axis_name="s")

    @functools.partial(
        pl.kernel, mesh=mesh,
        out_type=jax.ShapeDtypeStruct((T, HIDDEN), jnp.float32),
        scratch_types=[
            pltpu.VMEM((TPW // 2, HIDDEN), jnp.float32),
            pltpu.VMEM((TPW // 2, HIDDEN), jnp.float32),
            pltpu.VMEM((TPW // 2,), jnp.int32),
            pltpu.VMEM((TPW // 2,), jnp.int32),
            pltpu.SemaphoreType.DMA,
            pltpu.SemaphoreType.DMA,
        ],
    )
    def k(yg_hbm, dest_hbm, out_hbm, buf0_v, buf1_v, idx0_v, idx1_v,
          sem_a, sem_b):
        wid = lax.axis_index("s") * 2 + lax.axis_index("c")
        half = TPW // 2
        for h in range(2):
            base = wid * TPW + h * half
            pltpu.sync_copy(dest_hbm.at[0, pl.ds(base, half)], idx0_v)
            pltpu.sync_copy(dest_hbm.at[1, pl.ds(base, half)], idx1_v)
            c0 = pltpu.async_copy(yg_hbm.at[idx0_v], buf0_v, sem_a)
            c1 = pltpu.async_copy(yg_hbm.at[idx1_v], buf1_v, sem_b)
            c0.wait()
            c1.wait()

            def row(i, carry):
                for j in range(HIDDEN // 16):
                    sl = pl.ds(j * 16, 16)
                    buf0_v[i, sl] = buf0_v[i, sl] + buf1_v[i, sl]
                return carry

            lax.fori_loop(0, half, row, 0)
            pltpu.sync_copy(buf0_v, out_hbm.at[pl.ds(base, half)])

    return k


def kernel(hidden_states, gate_w, w_gate, w_up, w_down, num_global_tokens,
           max_num_tokens_per_gpu):
    del num_global_tokens, max_num_tokens_per_gpu
    dest, meta, wrep = _router_call(hidden_states, gate_w)
    xg, wgrp = _make_sc_scatter()(hidden_states, dest, wrep)
    yg = _ffn_call(meta, xg, wgrp, w_gate, w_up, w_down)
    return _make_sc_combine()(yg, dest)


# speedup vs baseline: 3.2726x; 1.0683x over previous
import functools

import jax
import jax.numpy as jnp
from jax import lax
from jax.experimental import pallas as pl
from jax.experimental.pallas import tpu as pltpu
from jax.experimental.pallas import tpu_sc as plsc

E = 16
TOP_K = 2
HIDDEN = 1024
INTER = 512
T = 2048
NEG_INF = float("-inf")

BLK = 512
NB = (T * TOP_K) // BLK + (E - 1)
GROUP_ROWS = NB * BLK
CHUNK = 256
NW = 32
TPW = T // NW
CC = 16
NCH = TPW // CC


def _router_body(x_ref, gate_ref, dest_ref, meta_ref, wts_ref):
    x = x_ref[...]
    logits = lax.dot_general(x, gate_ref[...], (((1,), (1,)), ((), ())),
                             preferred_element_type=jnp.float32)
    ii = lax.broadcasted_iota(jnp.int32, (T, E), 1)
    m1 = jnp.max(logits, axis=-1, keepdims=True)
    i1 = jnp.min(jnp.where(logits == m1, ii, E), axis=-1, keepdims=True)
    l2 = jnp.where(ii == i1, NEG_INF, logits)
    m2 = jnp.max(l2, axis=-1, keepdims=True)
    i2 = jnp.min(jnp.where(l2 == m2, ii, E), axis=-1, keepdims=True)
    r = jnp.exp(m2 - m1)
    w1 = 1.0 / (1.0 + r)
    w2 = 1.0 - w1
    wts_ref[0:T, :] = jnp.broadcast_to(w1, (T, 16))
    wts_ref[T:2 * T, :] = jnp.broadcast_to(w2, (T, 16))

    oh0 = (ii == i1).astype(jnp.float32)
    oh1 = (ii == i2).astype(jnp.float32)

    rr = lax.broadcasted_iota(jnp.int32, (CHUNK, CHUNK), 0)
    cc = lax.broadcasted_iota(jnp.int32, (CHUNK, CHUNK), 1)
    ltri = (rr > cc).astype(jnp.float32)

    def _excl_cumsum(oh):
        parts = []
        off = jnp.zeros((1, E), jnp.float32)
        for c in range(T // CHUNK):
            blk = oh[c * CHUNK:(c + 1) * CHUNK, :]
            exc = lax.dot_general(ltri, blk, (((1,), (0,)), ((), ())),
                                  preferred_element_type=jnp.float32)
            parts.append(exc + off)
            off = off + jnp.sum(blk, axis=0, keepdims=True)
        return jnp.concatenate(parts, axis=0), off

    exc0, cnt0 = _excl_cumsum(oh0)
    exc1, cnt1 = _excl_cumsum(oh1)
    counts = cnt0 + cnt1
    nb = jnp.floor((counts + (BLK - 1)) / BLK)
    er = lax.broadcasted_iota(jnp.int32, (E, E), 0)
    ec = lax.broadcasted_iota(jnp.int32, (E, E), 1)
    eutri = (er < ec).astype(jnp.float32)
    bs = lax.dot_general(nb, eutri, (((1,), (0,)), ((), ())),
                         preferred_element_type=jnp.float32)
    total = jnp.sum(nb)

    rank0 = jnp.sum(oh0 * exc0, axis=1)
    rank1 = jnp.sum(oh1 * (cnt0 + exc1), axis=1)
    base0 = jnp.sum(oh0 * bs, axis=1) * BLK
    base1 = jnp.sum(oh1 * bs, axis=1) * BLK
    dest_ref[0, :] = (base0 + rank0).astype(jnp.int32)
    dest_ref[1, :] = (base1 + rank1).astype(jnp.int32)

    bb = lax.broadcasted_iota(jnp.int32, (64, E), 0).astype(jnp.float32)
    emap = jnp.sum((bs <= bb).astype(jnp.int32), axis=1) - 1
    bidx = lax.broadcasted_iota(jnp.int32, (64, 1), 0).astype(jnp.float32)[:, 0]
    active = (bidx < total).astype(jnp.int32)
    meta_ref[0, :] = jnp.clip(emap, 0, E - 1)
    meta_ref[1, :] = active
    meta_ref[2, :] = jnp.minimum(bidx, total - 1.0).astype(jnp.int32)


def _router_call(x, gate_w):
    return pl.pallas_call(
        _router_body,
        out_shape=(
            jax.ShapeDtypeStruct((2, T), jnp.int32),
            jax.ShapeDtypeStruct((3, 64), jnp.int32),
            jax.ShapeDtypeStruct((2 * T, 16), jnp.float32),
        ),
    )(x, gate_w)


def _make_sc_scatter():
    mesh = plsc.VectorSubcoreMesh(core_axis_name="c", subcore_axis_name="s")

    @functools.partial(
        pl.kernel, mesh=mesh,
        out_type=jax.ShapeDtypeStruct((GROUP_ROWS, HIDDEN), jnp.float32),
        scratch_types=[
            pltpu.VMEM((TPW, HIDDEN), jnp.float32),
            pltpu.VMEM((TPW,), jnp.int32),
            pltpu.VMEM((TPW,), jnp.int32),
            pltpu.SemaphoreType.DMA,
            pltpu.SemaphoreType.DMA,
        ],
    )
    def k(x_hbm, dest_hbm, xg_hbm, rows_v, idx0_v, idx1_v, sem_a, sem_b):
        wid = lax.axis_index("s") * 2 + lax.axis_index("c")
        base = wid * TPW
        pltpu.sync_copy(dest_hbm.at[0, pl.ds(base, TPW)], idx0_v)
        pltpu.sync_copy(dest_hbm.at[1, pl.ds(base, TPW)], idx1_v)
        pltpu.sync_copy(x_hbm.at[pl.ds(base, TPW)], rows_v)
        c0 = pltpu.async_copy(rows_v, xg_hbm.at[idx0_v], sem_a)
        c1 = pltpu.async_copy(rows_v, xg_hbm.at[idx1_v], sem_b)
        c0.wait()
        c1.wait()

    return k


def _ffn_body(s_ref, xg_ref, wg_ref, wu_ref, wd_ref, yg_ref):
    b = pl.program_id(0)

    @pl.when(s_ref[1, b] == 1)
    def _compute():
        xb = xg_ref[...]
        g = lax.dot_general(xb, wg_ref[0], (((1,), (1,)), ((), ())),
                            preferred_element_type=jnp.float32)
        u = lax.dot_general(xb, wu_ref[0], (((1,), (1,)), ((), ())),
                            preferred_element_type=jnp.float32)
        h = (g * jax.nn.sigmoid(g)) * u
        yg_ref[...] = lax.dot_general(h, wd_ref[0], (((1,), (1,)), ((), ())),
                                      preferred_element_type=jnp.float32)


def _ffn_call(meta, xg, w_gate, w_up, w_down):
    grid_spec = pltpu.PrefetchScalarGridSpec(
        num_scalar_prefetch=1,
        grid=(NB,),
        in_specs=[
            pl.BlockSpec((BLK, HIDDEN), lambda b, s: (s[2, b], 0)),
            pl.BlockSpec((1, INTER, HIDDEN), lambda b, s: (s[0, b], 0, 0)),
            pl.BlockSpec((1, INTER, HIDDEN), lambda b, s: (s[0, b], 0, 0)),
            pl.BlockSpec((1, HIDDEN, INTER), lambda b, s: (s[0, b], 0, 0)),
        ],
        out_specs=pl.BlockSpec((BLK, HIDDEN), lambda b, s: (s[2, b], 0)),
    )
    return pl.pallas_call(
        _ffn_body,
        grid_spec=grid_spec,
        out_shape=jax.ShapeDtypeStruct((GROUP_ROWS, HIDDEN), jnp.float32),
    )(meta, xg, w_gate, w_up, w_down)


def _make_sc_combine():
    mesh = plsc.VectorSubcoreMesh(core_axis_name="c", subcore_axis_name="s")

    @functools.partial(
        pl.kernel, mesh=mesh,
        out_type=jax.ShapeDtypeStruct((T, HIDDEN), jnp.float32),
        scratch_types=[
            pltpu.VMEM((CC, HIDDEN), jnp.float32),
            pltpu.VMEM((CC, HIDDEN), jnp.float32),
            pltpu.VMEM((CC, HIDDEN), jnp.float32),
            pltpu.VMEM((CC, HIDDEN), jnp.float32),
            pltpu.VMEM((TPW,), jnp.int32),
            pltpu.VMEM((TPW,), jnp.int32),
            pltpu.VMEM((TPW, 16), jnp.float32),
            pltpu.VMEM((TPW, 16), jnp.float32),
            pltpu.VMEM((CC,), jnp.int32),
            pltpu.VMEM((CC,), jnp.int32),
            pltpu.VMEM((CC,), jnp.int32),
            pltpu.VMEM((CC,), jnp.int32),
            pltpu.SemaphoreType.DMA,
            pltpu.SemaphoreType.DMA,
            pltpu.SemaphoreType.DMA,
            pltpu.SemaphoreType.DMA,
            pltpu.SemaphoreType.DMA,
            pltpu.SemaphoreType.DMA,
        ],
    )
    def k(yg_hbm, dest_hbm, wts_hbm, out_hbm,
          a0_v, a1_v, b0_v, b1_v, idx0_v, idx1_v, w0_v, w1_v,
          ia0_v, ia1_v, ib0_v, ib1_v,
          sem_a0, sem_a1, sem_b0, sem_b1, sem_wa, sem_wb):
        wid = lax.axis_index("s") * 2 + lax.axis_index("c")
        base = wid * TPW
        pltpu.sync_copy(dest_hbm.at[0, pl.ds(base, TPW)], idx0_v)
        pltpu.sync_copy(dest_hbm.at[1, pl.ds(base, TPW)], idx1_v)
        pltpu.sync_copy(wts_hbm.at[pl.ds(base, TPW)], w0_v)
        pltpu.sync_copy(wts_hbm.at[pl.ds(T + base, TPW)], w1_v)

        bufs = [(a0_v, a1_v, ia0_v, ia1_v, sem_a0, sem_a1, sem_wa),
                (b0_v, b1_v, ib0_v, ib1_v, sem_b0, sem_b1, sem_wb)]

        def start(c):
            y0, y1, i0, i1, s0, s1, _ = bufs[c % 2]
            i0[...] = idx0_v[pl.ds(c * CC, CC)]
            i1[...] = idx1_v[pl.ds(c * CC, CC)]
            g0 = pltpu.async_copy(yg_hbm.at[i0], y0, s0)
            g1 = pltpu.async_copy(yg_hbm.at[i1], y1, s1)
            return g0, g1

        pend = start(0)
        wpend = [None, None]
        for c in range(NCH):
            y0, y1, _, _, _, _, sw = bufs[c % 2]
            pend[0].wait()
            pend[1].wait()
            if c + 1 < NCH:
                if wpend[(c + 1) % 2] is not None:
                    wpend[(c + 1) % 2].wait()
                    wpend[(c + 1) % 2] = None
                pend = start(c + 1)
            def row(i, carry):
                w0 = w0_v[c * CC + i, :]
                w1 = w1_v[c * CC + i, :]
                for j in range(HIDDEN // 16):
                    sl = pl.ds(j * 16, 16)
                    y0[i, sl] = y0[i, sl] * w0 + y1[i, sl] * w1
                return carry

            lax.fori_loop(0, CC, row, 0)
            st = pltpu.async_copy(y0, out_hbm.at[pl.ds(base + c * CC, CC)],
                                  sw)
            wpend[c % 2] = st
        for p in wpend:
            if p is not None:
                p.wait()

    return k


def kernel(hidden_states, gate_w, w_gate, w_up, w_down, num_global_tokens,
           max_num_tokens_per_gpu):
    del num_global_tokens, max_num_tokens_per_gpu
    dest, meta, wts = _router_call(hidden_states, gate_w)
    xg = _make_sc_scatter()(hidden_states, dest)
    yg = _ffn_call(meta, xg, w_gate, w_up, w_down)
    return _make_sc_combine()(yg, dest, wts)
